# baseline (speedup 1.0000x reference)
import jax
import jax.numpy as jnp
from jax import lax
from jax.experimental import pallas as pl
from jax.experimental.pallas import tpu as pltpu
from jax.experimental.pallas import tpu_sc as plsc

_T = 8
_LAT = 721
_LON = 1440
_LANES = 16
_NSUB = 16
_CHUNK = 48
_NCHUNK = _LON // _CHUNK


def _dyn_gather(x, perm):
    dnums = lax.GatherDimensionNumbers(
        offset_dims=(), collapsed_slice_dims=(0,), start_index_map=(0,))
    return lax.gather(x, perm[:, None], dnums, (1,),
                      mode=lax.GatherScatterMode.PROMISE_IN_BOUNDS)


def _coupler_body(times_hbm, newt_hbm, qt_hbm, buf_sst, buf_ice, new_sst,
                  new_ice, out, tv_v, misc_v, row0, row1, ldsem0, ldsem1,
                  stsem):
    pltpu.sync_copy(times_hbm, tv_v.at[pl.ds(0, _T)])
    pltpu.sync_copy(newt_hbm, misc_v.at[pl.ds(0, 1)])
    pltpu.sync_copy(qt_hbm, misc_v.at[pl.ds(8, 1)])
    lanes0 = lax.iota(jnp.int32, _LANES)
    t = tv_v[...]
    m = misc_v[...]
    shift = _dyn_gather(t, jnp.minimum(lanes0 + 1, _LANES - 1))
    nt = _dyn_gather(m, jnp.zeros((_LANES,), jnp.int32))
    q = _dyn_gather(m, jnp.full((_LANES,), 8, jnp.int32))
    times16 = jnp.where(lanes0 == (_T - 1), nt, shift)
    diff = jnp.where(lanes0 >= _T, jnp.float32(jnp.inf), jnp.abs(times16 - q))
    lanes = lax.iota(jnp.int32, _LANES)
    vals, args = diff, lanes
    for off in (1, 2, 4, 8):
        perm = lax.bitwise_and(lanes + off, _LANES - 1)
        ov = _dyn_gather(vals, perm)
        oa = _dyn_gather(args, perm)
        take = (ov < vals) | ((ov == vals) & (oa < args))
        vals = jnp.where(take, ov, vals)
        args = jnp.where(take, oa, args)
    idx = args[0]
    is_new = idx == (_T - 1)
    srow = idx + 1

    c = lax.axis_index("c")
    s = lax.axis_index("s")

    def field(buf, new, slot):
        k1 = s + _NSUB
        r0 = pl.multiple_of(s * _CHUNK, 8)
        r1 = pl.multiple_of(k1 * _CHUNK, 8)
        have1 = k1 < _NCHUNK

        def start_copy(r, sem):
            @pl.when(is_new)
            def _():
                pltpu.async_copy(new.at[pl.ds(r, _CHUNK)],
                                 out.at[slot, pl.ds(r, _CHUNK)], sem)

            @pl.when(jnp.logical_not(is_new))
            def _():
                pltpu.async_copy(buf.at[srow, pl.ds(r, _CHUNK)],
                                 out.at[slot, pl.ds(r, _CHUNK)], sem)

        start_copy(r0, ldsem0)

        @pl.when(have1)
        def _():
            start_copy(r1, ldsem1)

        pltpu.make_async_copy(new.at[pl.ds(r0, _CHUNK)],
                              out.at[slot, pl.ds(r0, _CHUNK)], ldsem0).wait()

        @pl.when(have1)
        def _():
            pltpu.make_async_copy(new.at[pl.ds(r1, _CHUNK)],
                                  out.at[slot, pl.ds(r1, _CHUNK)], ldsem1).wait()

    @pl.when(c == 0)
    def _():
        field(buf_sst, new_sst, 0)

    @pl.when(c == 1)
    def _():
        field(buf_ice, new_ice, 1)


def kernel(buf_sst, buf_ice, times_buf, new_sst, new_ice, new_time, query_time):
    sst_t = jnp.transpose(buf_sst, (0, 2, 1))
    ice_t = jnp.transpose(buf_ice, (0, 2, 1))
    nsst_t = jnp.transpose(new_sst, (1, 0))
    nice_t = jnp.transpose(new_ice, (1, 0))

    mesh = plsc.VectorSubcoreMesh(core_axis_name="c", subcore_axis_name="s")
    fn = pl.kernel(
        _coupler_body,
        mesh=mesh,
        out_type=jax.ShapeDtypeStruct((2, _LON, _LAT), jnp.float32),
        scratch_types=[
            pltpu.VMEM((_LANES,), jnp.float32),
            pltpu.VMEM((_LANES,), jnp.float32),
            pltpu.VMEM((_CHUNK, _LAT), jnp.float32),
            pltpu.VMEM((_CHUNK, _LAT), jnp.float32),
            pltpu.SemaphoreType.DMA,
            pltpu.SemaphoreType.DMA,
            pltpu.SemaphoreType.DMA,
        ],
    )
    out_t = fn(times_buf, new_time, query_time, sst_t, ice_t, nsst_t, nice_t)
    return jnp.transpose(out_t, (0, 2, 1))


# speedup vs baseline: 10.1362x; 10.1362x over previous
import jax
import jax.numpy as jnp
from jax import lax
from jax.experimental import pallas as pl
from jax.experimental.pallas import tpu as pltpu
from jax.experimental.pallas import tpu_sc as plsc

_T = 8
_LAT = 721
_LON = 1440
_LANES = 16
_NSUB = 16
_CHUNK = 16
_NCHUNK = _LON // _CHUNK
_NB = 3
_NJ = -(-_NCHUNK // _NSUB)
_LAST_S = _NCHUNK - _NSUB * (_NJ - 1)


def _dyn_gather(x, perm):
    dnums = lax.GatherDimensionNumbers(
        offset_dims=(), collapsed_slice_dims=(0,), start_index_map=(0,))
    return lax.gather(x, perm[:, None], dnums, (1,),
                      mode=lax.GatherScatterMode.PROMISE_IN_BOUNDS)


def _coupler_body(times_hbm, newt_hbm, qt_hbm, buf_sst, buf_ice, new_sst,
                  new_ice, out, tv_v, misc_v, row_a, row_b, row_c,
                  ld0, ld1, ld2, st0, st1, st2):
    rows = (row_a, row_b, row_c)
    lds = (ld0, ld1, ld2)
    sts = (st0, st1, st2)
    pltpu.sync_copy(times_hbm, tv_v.at[pl.ds(0, _T)])
    pltpu.sync_copy(newt_hbm, misc_v.at[pl.ds(0, 1)])
    pltpu.sync_copy(qt_hbm, misc_v.at[pl.ds(8, 1)])
    lanes0 = lax.iota(jnp.int32, _LANES)
    t = tv_v[...]
    m = misc_v[...]
    shift = _dyn_gather(t, jnp.minimum(lanes0 + 1, _LANES - 1))
    nt = _dyn_gather(m, jnp.zeros((_LANES,), jnp.int32))
    q = _dyn_gather(m, jnp.full((_LANES,), 8, jnp.int32))
    times16 = jnp.where(lanes0 == (_T - 1), nt, shift)
    diff = jnp.where(lanes0 >= _T, jnp.float32(jnp.inf), jnp.abs(times16 - q))
    lanes = lax.iota(jnp.int32, _LANES)
    vals, args = diff, lanes
    for off in (1, 2, 4, 8):
        perm = lax.bitwise_and(lanes + off, _LANES - 1)
        ov = _dyn_gather(vals, perm)
        oa = _dyn_gather(args, perm)
        take = (ov < vals) | ((ov == vals) & (oa < args))
        vals = jnp.where(take, ov, vals)
        args = jnp.where(take, oa, args)
    idx = args[0]
    is_new = idx == (_T - 1)
    srow = idx + 1

    c = lax.axis_index("c")
    s = lax.axis_index("s")

    def field(buf, new, slot):
        def r_of(j):
            return pl.multiple_of((s + _NSUB * j) * _CHUNK, 8)

        def start_load(j, b):
            r = r_of(j)

            @pl.when(is_new)
            def _():
                pltpu.async_copy(new.at[pl.ds(r, _CHUNK)], rows[b], lds[b])

            @pl.when(jnp.logical_not(is_new))
            def _():
                pltpu.async_copy(buf.at[srow, pl.ds(r, _CHUNK)], rows[b],
                                 lds[b])

        def wait_load(j, b):
            pltpu.make_async_copy(new.at[pl.ds(r_of(j), _CHUNK)], rows[b],
                                  lds[b]).wait()

        def start_store(j, b):
            pltpu.async_copy(rows[b], out.at[slot, pl.ds(r_of(j), _CHUNK)],
                             sts[b])

        def wait_store(j, b):
            pltpu.make_async_copy(rows[b],
                                  out.at[slot, pl.ds(r_of(j), _CHUNK)],
                                  sts[b]).wait()

        for b in range(_NB):
            start_load(b, b)
        for j in range(_NB):
            wait_load(j, j)
            start_store(j, j)
        for j in range(_NB, _NJ):
            b = j % _NB
            if j == _NJ - 1:
                @pl.when(s < _LAST_S)
                def _():
                    wait_store(j - _NB, b)
                    start_load(j, b)
            else:
                wait_store(j - _NB, b)
                start_load(j, b)
        for j in range(_NB, _NJ):
            b = j % _NB
            if j == _NJ - 1:
                @pl.when(s < _LAST_S)
                def _():
                    wait_load(j, b)
                    start_store(j, b)
            else:
                wait_load(j, b)
                start_store(j, b)
        for j in range(_NB, _NJ - 1):
            wait_store(j, j % _NB)
        lb = (_NJ - 1) % _NB

        @pl.when(s < _LAST_S)
        def _():
            wait_store(_NJ - 1, lb)

        @pl.when(s >= _LAST_S)
        def _():
            wait_store(_NJ - 1 - _NB, lb)

    @pl.when(c == 0)
    def _():
        field(buf_sst, new_sst, 0)

    @pl.when(c == 1)
    def _():
        field(buf_ice, new_ice, 1)


def kernel(buf_sst, buf_ice, times_buf, new_sst, new_ice, new_time, query_time):
    sst_t = jnp.transpose(buf_sst, (0, 2, 1))
    ice_t = jnp.transpose(buf_ice, (0, 2, 1))
    nsst_t = jnp.transpose(new_sst, (1, 0))
    nice_t = jnp.transpose(new_ice, (1, 0))

    mesh = plsc.VectorSubcoreMesh(core_axis_name="c", subcore_axis_name="s")
    fn = pl.kernel(
        _coupler_body,
        mesh=mesh,
        out_type=jax.ShapeDtypeStruct((2, _LON, _LAT), jnp.float32),
        scratch_types=(
            [pltpu.VMEM((_LANES,), jnp.float32)] * 2
            + [pltpu.VMEM((_CHUNK, _LAT), jnp.float32)] * _NB
            + [pltpu.SemaphoreType.DMA] * (2 * _NB)
        ),
    )
    out_t = fn(times_buf, new_time, query_time, sst_t, ice_t, nsst_t, nice_t)
    return jnp.transpose(out_t, (0, 2, 1))


# speedup vs baseline: 10.8310x; 1.0685x over previous
import jax
import jax.numpy as jnp
from jax import lax
from jax.experimental import pallas as pl
from jax.experimental.pallas import tpu as pltpu
from jax.experimental.pallas import tpu_sc as plsc

_T = 8
_LAT = 721
_LON = 1440
_LANES = 16
_NSUB = 16
_CHUNK = 48
_NCHUNK = _LON // _CHUNK


def _dyn_gather(x, perm):
    dnums = lax.GatherDimensionNumbers(
        offset_dims=(), collapsed_slice_dims=(0,), start_index_map=(0,))
    return lax.gather(x, perm[:, None], dnums, (1,),
                      mode=lax.GatherScatterMode.PROMISE_IN_BOUNDS)


def _coupler_body(times_hbm, newt_hbm, qt_hbm, buf_sst, buf_ice, new_sst,
                  new_ice, out, tv_v, misc_v, row0, row1, ldsem0, ldsem1,
                  stsem):
    cp_t = pltpu.make_async_copy(times_hbm, tv_v.at[pl.ds(0, _T)], ldsem0)
    cp_n = pltpu.make_async_copy(newt_hbm, misc_v.at[pl.ds(0, 1)], ldsem1)
    cp_q = pltpu.make_async_copy(qt_hbm, misc_v.at[pl.ds(8, 1)], stsem)
    cp_t.start()
    cp_n.start()
    cp_q.start()
    cp_t.wait()
    cp_n.wait()
    cp_q.wait()
    lanes0 = lax.iota(jnp.int32, _LANES)
    t = tv_v[...]
    m = misc_v[...]
    shift = _dyn_gather(t, jnp.minimum(lanes0 + 1, _LANES - 1))
    nt = _dyn_gather(m, jnp.zeros((_LANES,), jnp.int32))
    q = _dyn_gather(m, jnp.full((_LANES,), 8, jnp.int32))
    times16 = jnp.where(lanes0 == (_T - 1), nt, shift)
    diff = jnp.where(lanes0 >= _T, jnp.float32(jnp.inf), jnp.abs(times16 - q))
    lanes = lax.iota(jnp.int32, _LANES)
    vals, args = diff, lanes
    for off in (1, 2, 4, 8):
        perm = lax.bitwise_and(lanes + off, _LANES - 1)
        ov = _dyn_gather(vals, perm)
        oa = _dyn_gather(args, perm)
        take = (ov < vals) | ((ov == vals) & (oa < args))
        vals = jnp.where(take, ov, vals)
        args = jnp.where(take, oa, args)
    idx = args[0]
    is_new = idx == (_T - 1)
    srow = idx + 1

    c = lax.axis_index("c")
    s = lax.axis_index("s")

    def field(buf, new, slot):
        k1 = s + _NSUB
        r0 = pl.multiple_of(s * _CHUNK, 8)
        r1 = pl.multiple_of(k1 * _CHUNK, 8)
        have1 = k1 < _NCHUNK

        def start_load(r, dst, sem):
            @pl.when(is_new)
            def _():
                pltpu.async_copy(new.at[pl.ds(r, _CHUNK)], dst, sem)

            @pl.when(jnp.logical_not(is_new))
            def _():
                pltpu.async_copy(buf.at[srow, pl.ds(r, _CHUNK)], dst, sem)

        start_load(r0, row0, ldsem0)

        @pl.when(have1)
        def _():
            start_load(r1, row1, ldsem1)

        pltpu.make_async_copy(new.at[pl.ds(r0, _CHUNK)], row0, ldsem0).wait()
        pltpu.async_copy(row0, out.at[slot, pl.ds(r0, _CHUNK)], stsem)

        @pl.when(have1)
        def _():
            pltpu.make_async_copy(new.at[pl.ds(r1, _CHUNK)], row1, ldsem1).wait()
            pltpu.async_copy(row1, out.at[slot, pl.ds(r1, _CHUNK)], stsem)
            pltpu.make_async_copy(row1, out.at[slot, pl.ds(r1, _CHUNK)], stsem).wait()

        pltpu.make_async_copy(row0, out.at[slot, pl.ds(r0, _CHUNK)], stsem).wait()

    @pl.when(c == 0)
    def _():
        field(buf_sst, new_sst, 0)

    @pl.when(c == 1)
    def _():
        field(buf_ice, new_ice, 1)


def kernel(buf_sst, buf_ice, times_buf, new_sst, new_ice, new_time, query_time):
    sst_t = jnp.transpose(buf_sst, (0, 2, 1))
    ice_t = jnp.transpose(buf_ice, (0, 2, 1))
    nsst_t = jnp.transpose(new_sst, (1, 0))
    nice_t = jnp.transpose(new_ice, (1, 0))

    mesh = plsc.VectorSubcoreMesh(core_axis_name="c", subcore_axis_name="s")
    fn = pl.kernel(
        _coupler_body,
        mesh=mesh,
        out_type=jax.ShapeDtypeStruct((2, _LON, _LAT), jnp.float32),
        scratch_types=[
            pltpu.VMEM((_LANES,), jnp.float32),
            pltpu.VMEM((_LANES,), jnp.float32),
            pltpu.VMEM((_CHUNK, _LAT), jnp.float32),
            pltpu.VMEM((_CHUNK, _LAT), jnp.float32),
            pltpu.SemaphoreType.DMA,
            pltpu.SemaphoreType.DMA,
            pltpu.SemaphoreType.DMA,
        ],
    )
    out_t = fn(times_buf, new_time, query_time, sst_t, ice_t, nsst_t, nice_t)
    return jnp.transpose(out_t, (0, 2, 1))
